# baseline (speedup 1.0000x reference)
import jax
import jax.numpy as jnp
from jax import lax
from jax.experimental import pallas as pl
from jax.experimental.pallas import tpu as pltpu
from jax.experimental.pallas import tpu_sc as plsc

_EMBED_DIM = 32
_BATCH = 16384
_PACK = 4

_info = plsc.get_sparse_core_info()
_NC, _NS, _L = _info.num_cores, _info.num_subcores, _info.num_lanes
_NW = _NC * _NS
_B_PER_W = _BATCH // _NW
_CHUNK = 128
_N_CHUNKS = _B_PER_W // _CHUNK
_GROUPS = _CHUNK // _L


def _gather_body(idx_hbm, t128_hbm, out_hbm, idx_v, row_v, blk_v, rows_v,
                 out_v, sem):
    wid = lax.axis_index("s") * _NC + lax.axis_index("c")
    base = wid * _N_CHUNKS
    pltpu.sync_copy(idx_hbm.at[pl.ds(base, _N_CHUNKS)], idx_v)

    def precompute(i, _):
        j = i // _GROUPS
        b = i % _GROUPS
        v = idx_v[j, pl.ds(b * _L, _L)]
        row_v[j, pl.ds(b * _L, _L)] = lax.shift_right_logical(v, 2)
        blk_v[j, pl.ds(b * _L, _L)] = lax.mul(
            jnp.bitwise_and(v, 3), jnp.full((_L,), _EMBED_DIM, jnp.int32)
        )
        return 0

    lax.fori_loop(0, _N_CHUNKS * _GROUPS, precompute, 0)

    copies = [
        pltpu.async_copy(t128_hbm.at[row_v.at[j]], rows_v.at[j], sem)
        for j in range(_N_CHUNKS)
    ]

    lane_iota = lax.iota(jnp.int32, _L)

    for j in range(_N_CHUNKS):
        copies[j].wait()

        def select(i, _):
            b = i // _EMBED_DIM
            w = i % _EMBED_DIM
            local = lane_iota + b * _L
            cols = blk_v[j, pl.ds(b * _L, _L)] + w
            vals = plsc.load_gather(rows_v.at[j], [local, cols])
            plsc.store_scatter(
                out_v,
                [local + j * _CHUNK, jnp.full((_L,), w, jnp.int32)],
                vals,
            )
            return 0

        lax.fori_loop(0, _GROUPS * _EMBED_DIM, select, 0)

    pltpu.sync_copy(out_v, out_hbm.at[pl.ds(wid * _B_PER_W, _B_PER_W)])


def kernel(num_group, table):
    idx = num_group.astype(jnp.int32).reshape(_NW * _N_CHUNKS, _CHUNK)
    t128 = table.reshape(table.shape[0] // _PACK, _PACK * _EMBED_DIM)
    k = pl.kernel(
        _gather_body,
        out_type=jax.ShapeDtypeStruct((_BATCH, _EMBED_DIM), jnp.float32),
        mesh=plsc.VectorSubcoreMesh(core_axis_name="c", subcore_axis_name="s"),
        scratch_types=[
            pltpu.VMEM((_N_CHUNKS, _CHUNK), jnp.int32),
            pltpu.VMEM((_N_CHUNKS, _CHUNK), jnp.int32),
            pltpu.VMEM((_N_CHUNKS, _CHUNK), jnp.int32),
            pltpu.VMEM((_N_CHUNKS, _CHUNK, _PACK * _EMBED_DIM), jnp.float32),
            pltpu.VMEM((_B_PER_W, _EMBED_DIM), jnp.float32),
            pltpu.SemaphoreType.DMA,
        ],
        compiler_params=pltpu.CompilerParams(
            use_tc_tiling_on_sc=False, needs_layout_passes=False
        ),
    )
    return k(idx, t128)


# speedup vs baseline: 1.0065x; 1.0065x over previous
import jax
import jax.numpy as jnp
from jax import lax
from jax.experimental import pallas as pl
from jax.experimental.pallas import tpu as pltpu
from jax.experimental.pallas import tpu_sc as plsc

_EMBED_DIM = 32
_BATCH = 16384
_PACK = 4

_info = plsc.get_sparse_core_info()
_NC, _NS, _L = _info.num_cores, _info.num_subcores, _info.num_lanes
_NW = _NC * _NS
_B_PER_W = _BATCH // _NW
_CHUNK = 128
_N_CHUNKS = _B_PER_W // _CHUNK
_GROUPS = _CHUNK // _L


def _gather_body(idx_hbm, t128_hbm, out_hbm, idx_v, row_v, blk_v, rows_v,
                 out_v, sem):
    wid = lax.axis_index("s") * _NC + lax.axis_index("c")
    base = wid * _N_CHUNKS
    pltpu.sync_copy(idx_hbm.at[pl.ds(base, _N_CHUNKS)], idx_v)

    def precompute(i, _):
        j = i // _GROUPS
        b = i % _GROUPS
        v = idx_v[j, pl.ds(b * _L, _L)]
        row_v[j, pl.ds(b * _L, _L)] = lax.shift_right_logical(v, 2)
        blk_v[j, pl.ds(b * _L, _L)] = lax.mul(
            jnp.bitwise_and(v, 3), jnp.full((_L,), _EMBED_DIM, jnp.int32)
        )
        return 0

    lax.fori_loop(0, _N_CHUNKS * _GROUPS, precompute, 0)

    copies = {}
    for j in range(2):
        copies[j] = pltpu.async_copy(
            t128_hbm.at[row_v.at[j]], rows_v.at[j % 2], sem
        )

    lane_iota = lax.iota(jnp.int32, _L)

    for j in range(_N_CHUNKS):
        copies[j].wait()

        def select(i, _, j=j):
            b = i // _EMBED_DIM
            w = i % _EMBED_DIM
            local = lane_iota + b * _L
            cols = blk_v[j, pl.ds(b * _L, _L)] + w
            vals = plsc.load_gather(rows_v.at[j % 2], [local, cols])
            plsc.store_scatter(
                out_v,
                [local + j * _CHUNK, jnp.full((_L,), w, jnp.int32)],
                vals,
            )
            return 0

        lax.fori_loop(0, _GROUPS * _EMBED_DIM, select, 0)
        if j + 2 < _N_CHUNKS:
            copies[j + 2] = pltpu.async_copy(
                t128_hbm.at[row_v.at[j + 2]], rows_v.at[j % 2], sem
            )

    pltpu.sync_copy(out_v, out_hbm.at[pl.ds(wid * _B_PER_W, _B_PER_W)])


def kernel(num_group, table):
    idx = num_group.astype(jnp.int32).reshape(_NW * _N_CHUNKS, _CHUNK)
    t128 = table.reshape(table.shape[0] // _PACK, _PACK * _EMBED_DIM)
    k = pl.kernel(
        _gather_body,
        out_type=jax.ShapeDtypeStruct((_BATCH, _EMBED_DIM), jnp.float32),
        mesh=plsc.VectorSubcoreMesh(core_axis_name="c", subcore_axis_name="s"),
        scratch_types=[
            pltpu.VMEM((_N_CHUNKS, _CHUNK), jnp.int32),
            pltpu.VMEM((_N_CHUNKS, _CHUNK), jnp.int32),
            pltpu.VMEM((_N_CHUNKS, _CHUNK), jnp.int32),
            pltpu.VMEM((2, _CHUNK, _PACK * _EMBED_DIM), jnp.float32),
            pltpu.VMEM((_B_PER_W, _EMBED_DIM), jnp.float32),
            pltpu.SemaphoreType.DMA,
        ],
        compiler_params=pltpu.CompilerParams(
            use_tc_tiling_on_sc=True, needs_layout_passes=False
        ),
    )
    return k(idx, t128)


# speedup vs baseline: 1.0503x; 1.0436x over previous
import jax
import jax.numpy as jnp
from jax import lax
from jax.experimental import pallas as pl
from jax.experimental.pallas import tpu as pltpu
from jax.experimental.pallas import tpu_sc as plsc

_EMBED_DIM = 32
_BATCH = 16384

_info = plsc.get_sparse_core_info()
_NC, _NS = _info.num_cores, _info.num_subcores
_NW = _NC * _NS
_CHUNK = 128
_B_PER_W = _BATCH // _NW
_N_CHUNKS = _B_PER_W // _CHUNK


def _gather_body(idx_hbm, table_hbm, out_hbm, idx_v, rows_v, sem):
    wid = lax.axis_index("s") * _NC + lax.axis_index("c")
    base = wid * _N_CHUNKS
    pltpu.sync_copy(idx_hbm.at[pl.ds(base, _N_CHUNKS)], idx_v)
    copies = [
        pltpu.async_copy(table_hbm.at[idx_v.at[j]], rows_v.at[j], sem)
        for j in range(_N_CHUNKS)
    ]
    for c in copies:
        c.wait()
    pltpu.sync_copy(rows_v, out_hbm.at[pl.ds(base, _N_CHUNKS)])


def kernel(num_group, table):
    idx = num_group.astype(jnp.int32).reshape(_NW * _N_CHUNKS, _CHUNK)
    k = pl.kernel(
        _gather_body,
        out_type=jax.ShapeDtypeStruct((_NW * _N_CHUNKS, _CHUNK, _EMBED_DIM),
                                      jnp.float32),
        mesh=plsc.VectorSubcoreMesh(core_axis_name="c", subcore_axis_name="s"),
        scratch_types=[
            pltpu.VMEM((_N_CHUNKS, _CHUNK), jnp.int32),
            pltpu.VMEM((_N_CHUNKS, _CHUNK, _EMBED_DIM), jnp.float32),
            pltpu.SemaphoreType.DMA,
        ],
        compiler_params=pltpu.CompilerParams(use_tc_tiling_on_sc=False),
    )
    out = k(idx, table)
    return out.reshape(_BATCH, _EMBED_DIM)


# speedup vs baseline: 1.3738x; 1.3081x over previous
import jax
import jax.numpy as jnp
from jax import lax
from jax.experimental import pallas as pl
from jax.experimental.pallas import tpu as pltpu
from jax.experimental.pallas import tpu_sc as plsc

_EMBED_DIM = 32
_BATCH = 16384
_ROWS = 1000000

_info = plsc.get_sparse_core_info()
_NC, _NS, _L = _info.num_cores, _info.num_subcores, _info.num_lanes
_NW = _NC * _NS
_CH = 512
_CH_SHIFT = 9
_NFULL = 1952
_TAIL0 = _NFULL * _CH
_TAIL1 = _TAIL0 + _CH
_TAIL1_LEN = _ROWS - _TAIL1
_IDX_VREGS = _BATCH // _L


def _body(idx_hbm, tt_hbm, tail_hbm, out_hbm, idx_v, rlist, blist, buf,
          rowstage, sem_s, sem_o):
    w = lax.axis_index("s") * _NC + lax.axis_index("c")
    pltpu.sync_copy(idx_hbm, idx_v)

    lane_iota = lax.iota(jnp.int32, _L)
    sentinel = jnp.full((_L,), jnp.int32(2**30), jnp.int32)

    plsc.store_scatter(rlist, [lane_iota], sentinel)

    def filt(i, base):
        v = idx_v[pl.ds(i * _L, _L)]
        mask = jnp.bitwise_and(
            lax.shift_right_logical(v, _CH_SHIFT), jnp.int32(_NW - 1)
        ) == jnp.full((_L,), jnp.int32(0), jnp.int32) + w
        mi = jnp.where(mask, jnp.int32(1), jnp.int32(0))
        rank = plsc.cumsum(mi) - 1
        pos = base + rank
        plsc.store_scatter(rlist, [pos], v, mask=mask)
        plsc.store_scatter(blist, [pos], lane_iota + i * _L, mask=mask)
        cnt = plsc.all_reduce_population_count(mask)
        new_base = base + cnt
        plsc.store_scatter(rlist, [new_base + lane_iota], sentinel)
        return new_base

    base = lax.fori_loop(
        0, _IDX_VREGS, filt, jnp.zeros((_L,), jnp.int32)
    )
    count = jnp.max(base)
    nv = (count + jnp.int32(_L - 1)) // jnp.int32(_L)

    def process_chunk(lb, clen):
        def scan_list(q, _):
            r16 = rlist[pl.ds(q * _L, _L)]
            b16 = blist[pl.ds(q * _L, _L)]
            inm = jnp.logical_and(r16 >= lb, r16 < lb + clen)

            @pl.when(jnp.any(inm))
            def _():
                local = jnp.where(inm, r16 - lb, jnp.int32(0))
                for s in range(_EMBED_DIM):
                    svec = jnp.full((_L,), jnp.int32(s), jnp.int32)
                    vals = plsc.load_gather(buf, [svec, local])
                    plsc.store_scatter(rowstage, [lane_iota, svec], vals)
                bm = jnp.where(inm, b16, jnp.int32(-1))
                pltpu.async_copy(
                    rowstage,
                    out_hbm.at[plsc.Indices(bm, ignored_value=-1)],
                    sem_o,
                ).wait()

            return 0

        lax.fori_loop(0, nv, scan_list, 0)

    def main(t, _):
        m = w + t * _NW
        lb = pl.multiple_of(m * _CH, _CH)
        pltpu.sync_copy(tt_hbm.at[:, pl.ds(lb, _CH)], buf)
        process_chunk(lb, jnp.int32(_CH))
        return 0

    lax.fori_loop(0, _NFULL // _NW, main, 0)

    @pl.when(w == 0)
    def _():
        pltpu.sync_copy(tt_hbm.at[:, pl.ds(_TAIL0, _CH)], buf)
        process_chunk(jnp.int32(_TAIL0), jnp.int32(_CH))

    @pl.when(w == 1)
    def _():
        pltpu.sync_copy(tail_hbm, buf.at[:, pl.ds(0, 128)])
        process_chunk(jnp.int32(_TAIL1), jnp.int32(_TAIL1_LEN))


def kernel(num_group, table):
    idx = num_group.astype(jnp.int32)
    tt = table.T
    tail = jnp.pad(table[_TAIL1:], ((0, 128 - _TAIL1_LEN), (0, 0))).T
    k = pl.kernel(
        _body,
        out_type=jax.ShapeDtypeStruct((_BATCH, 128), jnp.float32),
        mesh=plsc.VectorSubcoreMesh(core_axis_name="c", subcore_axis_name="s"),
        scratch_types=[
            pltpu.VMEM((_BATCH,), jnp.int32),
            pltpu.VMEM((_BATCH + _L,), jnp.int32),
            pltpu.VMEM((_BATCH,), jnp.int32),
            pltpu.VMEM((_EMBED_DIM, _CH), jnp.float32),
            pltpu.VMEM((_L, 128), jnp.float32),
            pltpu.SemaphoreType.DMA,
            pltpu.SemaphoreType.DMA,
        ],
        compiler_params=pltpu.CompilerParams(
            use_tc_tiling_on_sc=True, needs_layout_passes=False
        ),
    )
    return k(idx, tt, tail)[:, :_EMBED_DIM]


# speedup vs baseline: 1.9722x; 1.4356x over previous
import jax
import jax.numpy as jnp
from jax import lax
from jax.experimental import pallas as pl
from jax.experimental.pallas import tpu as pltpu
from jax.experimental.pallas import tpu_sc as plsc

_EMBED_DIM = 32
_BATCH = 16384
_ROWS = 1000000

_info = plsc.get_sparse_core_info()
_NC, _NS, _L = _info.num_cores, _info.num_subcores, _info.num_lanes
_NW = _NC * _NS
_CH = 1024
_CH_SHIFT = 10
_NFULL = 976
_TAIL0 = _NFULL * _CH
_TAIL1 = _TAIL0 + 512
_TAIL1_LEN = _ROWS - _TAIL1
_IDX_VREGS = _BATCH // _L


def _body(idx_hbm, tt_hbm, tail_hbm, out_hbm, idx_v, rlist, blist, buf,
          rowstage, sem_s, sem_o):
    w = lax.axis_index("s") * _NC + lax.axis_index("c")
    pltpu.sync_copy(idx_hbm, idx_v)

    lane_iota = lax.iota(jnp.int32, _L)
    sentinel = jnp.full((_L,), jnp.int32(2**30), jnp.int32)

    plsc.store_scatter(rlist, [lane_iota], sentinel)

    def filt(i, base):
        v = idx_v[pl.ds(i * _L, _L)]
        mask = jnp.bitwise_and(
            lax.shift_right_logical(v, _CH_SHIFT), jnp.int32(_NW - 1)
        ) == jnp.full((_L,), jnp.int32(0), jnp.int32) + w
        mi = jnp.where(mask, jnp.int32(1), jnp.int32(0))
        rank = plsc.cumsum(mi) - 1
        pos = base + rank
        plsc.store_scatter(rlist, [pos], v, mask=mask)
        plsc.store_scatter(blist, [pos], lane_iota + i * _L, mask=mask)
        cnt = plsc.all_reduce_population_count(mask)
        new_base = base + cnt
        plsc.store_scatter(rlist, [new_base + lane_iota], sentinel)
        return new_base

    base = lax.fori_loop(
        0, _IDX_VREGS, filt, jnp.zeros((_L,), jnp.int32)
    )
    count = jnp.max(base)
    nv = (count + jnp.int32(_L - 1)) // jnp.int32(_L)

    def process_chunk(lb, clen, cbuf):
        def scan_list(q, _):
            r16 = rlist[pl.ds(q * _L, _L)]
            b16 = blist[pl.ds(q * _L, _L)]
            inm = jnp.logical_and(r16 >= lb, r16 < lb + clen)

            @pl.when(jnp.any(inm))
            def _():
                local = jnp.where(inm, r16 - lb, jnp.int32(0))
                for s in range(_EMBED_DIM):
                    svec = jnp.full((_L,), jnp.int32(s), jnp.int32)
                    vals = plsc.load_gather(cbuf, [svec, local])
                    plsc.store_scatter(rowstage, [lane_iota, svec], vals)
                bm = jnp.where(inm, b16, jnp.int32(-1))
                pltpu.async_copy(
                    rowstage,
                    out_hbm.at[plsc.Indices(bm, ignored_value=-1)],
                    sem_o,
                ).wait()

            return 0

        lax.fori_loop(0, nv, scan_list, 0)

    def valid(t):
        return (w + t * _NW) <= (_NFULL - 1)

    def lbof(t):
        return pl.multiple_of((w + t * _NW) * _CH, _CH)

    def fire(t, p):
        @pl.when(valid(t))
        def _():
            pltpu.async_copy(tt_hbm.at[:, pl.ds(lbof(t), _CH)],
                             buf.at[p], sem_s)

    def step(t, p):
        @pl.when(valid(t))
        def _():
            pltpu.make_async_copy(
                tt_hbm.at[:, pl.ds(0, _CH)], buf.at[p], sem_s
            ).wait()

        fire(t + 1, 1 - p)

        @pl.when(valid(t))
        def _():
            process_chunk(lbof(t), jnp.int32(_CH), buf.at[p])

    fire(0, 0)

    def pair(q, _):
        step(2 * q, 0)
        step(2 * q + 1, 1)
        return 0

    lax.fori_loop(0, (_NFULL // _NW + 2) // 2, pair, 0)

    @pl.when(w == 16)
    def _():
        pltpu.sync_copy(tt_hbm.at[:, pl.ds(_TAIL0, 512)],
                        buf.at[0].at[:, pl.ds(0, 512)])
        process_chunk(jnp.int32(_TAIL0), jnp.int32(512), buf.at[0])
        pltpu.sync_copy(tail_hbm, buf.at[0].at[:, pl.ds(0, 128)])
        process_chunk(jnp.int32(_TAIL1), jnp.int32(_TAIL1_LEN), buf.at[0])


def kernel(num_group, table):
    idx = num_group.astype(jnp.int32)
    tt = table.T
    tail = jnp.pad(table[_TAIL1:], ((0, 128 - _TAIL1_LEN), (0, 0))).T
    k = pl.kernel(
        _body,
        out_type=jax.ShapeDtypeStruct((_BATCH, 128), jnp.float32),
        mesh=plsc.VectorSubcoreMesh(core_axis_name="c", subcore_axis_name="s"),
        scratch_types=[
            pltpu.VMEM((_BATCH,), jnp.int32),
            pltpu.VMEM((_BATCH + _L,), jnp.int32),
            pltpu.VMEM((_BATCH,), jnp.int32),
            pltpu.VMEM((2, _EMBED_DIM, _CH), jnp.float32),
            pltpu.VMEM((_L, 128), jnp.float32),
            pltpu.SemaphoreType.DMA,
            pltpu.SemaphoreType.DMA,
        ],
        compiler_params=pltpu.CompilerParams(
            use_tc_tiling_on_sc=True, needs_layout_passes=False
        ),
    )
    return k(idx, tt, tail)[:, :_EMBED_DIM]
